# Initial kernel scaffold; baseline (speedup 1.0000x reference)
import jax
import jax.numpy as jnp
from jax.experimental import pallas as pl


def kernel(features, edge_index, edge_index_homo1, edge_index_homo2, W1, b1, W2, b2, W_dec, b_dec, attn_simplex, attn_complex):
    raise NotImplementedError("write your pallas kernel here")


# speedup vs baseline: 1.0873x; 1.0873x over previous
import functools
import jax
import jax.numpy as jnp
from jax.experimental import pallas as pl
from jax.experimental.pallas import tpu as pltpu

N = 10000
E = 320000
D = 128
NCLS = 16


def _agg(h, src, dst, coef):
    msgs = h[src] * coef[:, None]
    return jax.ops.segment_sum(msgs, dst, num_segments=N)


def _matmul_bias_kernel(x_ref, w_ref, b_ref, o_ref, *, relu_in):
    x = x_ref[...]
    if relu_in:
        x = jnp.maximum(x, 0.0)
    o_ref[...] = x @ w_ref[...] + b_ref[...]


def _matmul_bias(x, w, b, relu_in=False):
    n, d = x.shape
    dout = w.shape[1]
    return pl.pallas_call(
        functools.partial(_matmul_bias_kernel, relu_in=relu_in),
        out_shape=jax.ShapeDtypeStruct((n, dout), jnp.float32),
    )(x, w, b)


def _fuse_kernel(h_ref, h1_ref, h2_ref, as_ref, ac_ref, wd_ref, bd_ref, o_ref):
    h = jnp.maximum(h_ref[...], 0.0)
    h1 = jnp.maximum(h1_ref[...], 0.0)
    h2 = jnp.maximum(h2_ref[...], 0.0)
    p1 = jnp.max(h1, axis=0, keepdims=True)
    p2 = jnp.max(h2, axis=0, keepdims=True)
    l1 = jnp.sum(p1 * as_ref[...].T)
    l2 = jnp.sum(p2 * as_ref[...].T)
    m = jnp.maximum(l1, l2)
    e1 = jnp.exp(l1 - m)
    e2 = jnp.exp(l2 - m)
    a1 = e1 / (e1 + e2)
    a2 = e2 / (e1 + e2)
    h_homo = h1 * a1 + h2 * a2
    q1 = jnp.max(h, axis=0, keepdims=True)
    q2 = jnp.max(h_homo, axis=0, keepdims=True)
    c1 = jnp.sum(q1 * ac_ref[...].T)
    c2 = jnp.sum(q2 * ac_ref[...].T)
    mc = jnp.maximum(c1, c2)
    f1 = jnp.exp(c1 - mc)
    f2 = jnp.exp(c2 - mc)
    b1a = f1 / (f1 + f2)
    b2a = f2 / (f1 + f2)
    hf = h * b1a + h_homo * b2a
    o_ref[...] = hf @ wd_ref[...] + bd_ref[...]


def _fuse(h, h1, h2, attn_s, attn_c, W_dec, b_dec):
    return pl.pallas_call(
        _fuse_kernel,
        out_shape=jax.ShapeDtypeStruct((N, NCLS), jnp.float32),
    )(h, h1, h2, attn_s, attn_c, W_dec, b_dec)


def kernel(features, edge_index, edge_index_homo1, edge_index_homo2,
           W1, b1, W2, b2, W_dec, b_dec, attn_simplex, attn_complex):
    graphs = [(edge_index[0], edge_index[1]),
              (edge_index_homo1[0], edge_index_homo1[1]),
              (edge_index_homo2[0], edge_index_homo2[1])]
    coefs = []
    for src, dst in graphs:
        deg = jax.ops.segment_sum(jnp.ones((E,), jnp.float32), dst, num_segments=N)
        norm = jax.lax.rsqrt(jnp.clip(deg, 1.0, None))
        coefs.append(norm[src] * norm[dst])

    t0 = _matmul_bias(features, W1, b1)
    hs = []
    for (src, dst), coef in zip(graphs, coefs):
        a1 = _agg(t0, src, dst, coef)
        u = _matmul_bias(a1, W2, b2, relu_in=True)
        a2 = _agg(u, src, dst, coef)
        hs.append(a2)
    return _fuse(hs[0], hs[1], hs[2], attn_simplex, attn_complex, W_dec, b_dec)


# speedup vs baseline: 9.6217x; 8.8490x over previous
import functools
import jax
import jax.numpy as jnp
from jax import lax
from jax.experimental import pallas as pl
from jax.experimental.pallas import tpu as pltpu
from jax.experimental.pallas import tpu_sc as plsc

N = 10000
E = 320000
D = 128
NCLS = 16
NG = 3

NC = 2
NS = 16
NW = NC * NS
L = 16

K = 80
EPT = E // NW
NWIN = EPT // K
DEG_PAD = 30720
DEG_PT = DEG_PAD // NS
NP = 10240
ROWS_PT = NP // NS
CH = 128

_mesh = plsc.VectorSubcoreMesh(core_axis_name="c", subcore_axis_name="s")


@functools.partial(
    pl.kernel,
    out_type=jax.ShapeDtypeStruct((NC * DEG_PAD,), jnp.float32),
    mesh=_mesh,
    scratch_types=[
        pltpu.VMEM((K,), jnp.int32),
        pltpu.VMEM((K,), jnp.float32),
        pltpu.VMEM((DEG_PT,), jnp.float32),
        pltpu.VMEM_SHARED((DEG_PAD,), jnp.float32),
        pltpu.SemaphoreType.DMA,
    ],
)
def _deg_sc(dstb_hbm, out_hbm, idx_v, ones_v, stage_v, acc_sh, sem):
    c = lax.axis_index("c")
    s = lax.axis_index("s")
    w = c * NS + s
    for i in range(K // L):
        ones_v[pl.ds(i * L, L)] = jnp.full((L,), 1.0, jnp.float32)

    def zbody(i, carry):
        stage_v[pl.ds(i * L, L)] = jnp.zeros((L,), jnp.float32)
        return carry

    lax.fori_loop(0, DEG_PT // L, zbody, 0)
    pltpu.sync_copy(stage_v, acc_sh.at[pl.ds(s * DEG_PT, DEG_PT)])
    plsc.subcore_barrier()

    ept_all = NG * E // NW

    def body(win, carry):
        base = w * ept_all + win * K
        pltpu.sync_copy(dstb_hbm.at[pl.ds(base, K)], idx_v)
        pltpu.sync_copy(ones_v, acc_sh.at[idx_v], add=True)
        return carry

    lax.fori_loop(0, ept_all // K, body, 0)
    plsc.subcore_barrier()
    pltpu.sync_copy(acc_sh.at[pl.ds(s * DEG_PT, DEG_PT)], stage_v)
    pltpu.sync_copy(stage_v, out_hbm.at[pl.ds(c * DEG_PAD + s * DEG_PT, DEG_PT)])


@functools.partial(
    pl.kernel,
    out_type=jax.ShapeDtypeStruct((NG, NC, NP, D), jnp.float32),
    mesh=_mesh,
    scratch_types=[
        pltpu.VMEM((K,), jnp.int32),
        pltpu.VMEM((K,), jnp.int32),
        pltpu.VMEM((K, D), jnp.float32),
        pltpu.VMEM((CH, D), jnp.float32),
        pltpu.VMEM((CH, D), jnp.float32),
        pltpu.VMEM_SHARED((NP, D), jnp.float32),
        pltpu.SemaphoreType.DMA,
    ],
)
def _agg3_sc(tables_hbm, src_hbm, dst_hbm, out_hbm,
             idx_s, idx_d, rows_v, zbuf, wbuf, acc_sh, sem):
    c = lax.axis_index("c")
    s = lax.axis_index("s")
    w = c * NS + s

    def zb(i, carry):
        for j in range(D // L):
            zbuf[i, pl.ds(j * L, L)] = jnp.zeros((L,), jnp.float32)
        return carry

    lax.fori_loop(0, CH, zb, 0)
    for k in range(ROWS_PT // CH):
        pltpu.sync_copy(zbuf, acc_sh.at[pl.ds(s * ROWS_PT + k * CH, CH)])

    for g in range(NG):
        plsc.subcore_barrier()

        def body(win, carry, g=g):
            base = g * E + w * EPT + win * K
            pltpu.sync_copy(src_hbm.at[pl.ds(base, K)], idx_s)
            pltpu.sync_copy(dst_hbm.at[pl.ds(base, K)], idx_d)
            pltpu.async_copy(tables_hbm.at[g].at[idx_s], rows_v, sem).wait()
            pltpu.sync_copy(rows_v, acc_sh.at[idx_d], add=True)
            return carry

        lax.fori_loop(0, NWIN, body, 0)
        plsc.subcore_barrier()
        for k in range(ROWS_PT // CH):
            r0 = s * ROWS_PT + k * CH
            pltpu.sync_copy(acc_sh.at[pl.ds(r0, CH)], wbuf)
            pltpu.sync_copy(wbuf, out_hbm.at[g, c, pl.ds(r0, CH)])
            pltpu.sync_copy(zbuf, acc_sh.at[pl.ds(r0, CH)])


def _norm_kernel(deg_ref, o_ref):
    d = deg_ref[pl.ds(0, DEG_PAD)] + deg_ref[pl.ds(DEG_PAD, DEG_PAD)]
    o_ref[...] = lax.rsqrt(jnp.clip(d, 1.0, None))


def _norm(deg_partials):
    return pl.pallas_call(
        _norm_kernel,
        out_shape=jax.ShapeDtypeStruct((DEG_PAD,), jnp.float32),
    )(deg_partials)


def _mm1_kernel(x_ref, w_ref, b_ref, n_ref, o_ref):
    t0 = x_ref[...] @ w_ref[...] + b_ref[...]
    for g in range(NG):
        o_ref[g] = t0 * n_ref[g][:, None]


def _mm1(x, W1, b1, norm):
    return pl.pallas_call(
        _mm1_kernel,
        out_shape=jax.ShapeDtypeStruct((NG, N, D), jnp.float32),
    )(x, W1, b1, norm)


def _mm2_kernel(p_ref, w_ref, b_ref, n_ref, o_ref):
    ng = n_ref[0, 0][:, None]
    p = p_ref[0, 0][:N] + p_ref[0, 1][:N]
    h = jnp.maximum(p * ng, 0.0)
    o_ref[0] = (h @ w_ref[...] + b_ref[...]) * ng


def _mm2(partials, W2, b2, norm):
    return pl.pallas_call(
        _mm2_kernel,
        grid=(NG,),
        in_specs=[
            pl.BlockSpec((1, NC, NP, D), lambda g: (g, 0, 0, 0)),
            pl.BlockSpec((D, D), lambda g: (0, 0)),
            pl.BlockSpec((D,), lambda g: (0,)),
            pl.BlockSpec((1, 1, N), lambda g: (g, 0, 0)),
        ],
        out_specs=pl.BlockSpec((1, N, D), lambda g: (g, 0, 0)),
        out_shape=jax.ShapeDtypeStruct((NG, N, D), jnp.float32),
    )(partials, W2, b2, norm.reshape(NG, 1, N))


def _fuse_kernel(p_ref, n_ref, as_ref, ac_ref, wd_ref, bd_ref, o_ref):
    hs = []
    for g in range(NG):
        p = p_ref[g, 0][:N] + p_ref[g, 1][:N]
        hs.append(jnp.maximum(p * n_ref[g][:, None], 0.0))
    h, h1, h2 = hs
    l1 = jnp.sum(jnp.max(h1, axis=0) * as_ref[:, 0])
    l2 = jnp.sum(jnp.max(h2, axis=0) * as_ref[:, 0])
    m = jnp.maximum(l1, l2)
    e1 = jnp.exp(l1 - m)
    e2 = jnp.exp(l2 - m)
    h_homo = h1 * (e1 / (e1 + e2)) + h2 * (e2 / (e1 + e2))
    c1 = jnp.sum(jnp.max(h, axis=0) * ac_ref[:, 0])
    c2 = jnp.sum(jnp.max(h_homo, axis=0) * ac_ref[:, 0])
    mc = jnp.maximum(c1, c2)
    f1 = jnp.exp(c1 - mc)
    f2 = jnp.exp(c2 - mc)
    hf = h * (f1 / (f1 + f2)) + h_homo * (f2 / (f1 + f2))
    o_ref[...] = hf @ wd_ref[...] + bd_ref[...]


def _fuse(partials, norm, attn_s, attn_c, W_dec, b_dec):
    return pl.pallas_call(
        _fuse_kernel,
        out_shape=jax.ShapeDtypeStruct((N, NCLS), jnp.float32),
    )(partials, norm, attn_s, attn_c, W_dec, b_dec)


def kernel(features, edge_index, edge_index_homo1, edge_index_homo2,
           W1, b1, W2, b2, W_dec, b_dec, attn_simplex, attn_complex):
    src = jnp.stack([edge_index[0], edge_index_homo1[0], edge_index_homo2[0]])
    dst = jnp.stack([edge_index[1], edge_index_homo1[1], edge_index_homo2[1]])
    bias = (jnp.arange(NG, dtype=jnp.int32) * N)[:, None]
    dstb = (dst + bias).reshape(-1)
    deg_p = _deg_sc(dstb)
    norm_flat = _norm(deg_p)
    norm = norm_flat[:NG * N].reshape(NG, N)

    src_f = src.reshape(-1)
    dst_f = dst.reshape(-1)
    tables1 = _mm1(features, W1, b1, norm)
    p1 = _agg3_sc(tables1, src_f, dst_f)
    tables2 = _mm2(p1, W2, b2, norm)
    p2 = _agg3_sc(tables2, src_f, dst_f)
    return _fuse(p2, norm, attn_simplex, attn_complex, W_dec, b_dec)


# speedup vs baseline: 14.3164x; 1.4879x over previous
import functools
import jax
import jax.numpy as jnp
from jax import lax
from jax.experimental import pallas as pl
from jax.experimental.pallas import tpu as pltpu
from jax.experimental.pallas import tpu_sc as plsc

N = 10000
E = 320000
D = 128
NCLS = 16
NG = 3

NC = 2
NS = 16
NW = NC * NS
L = 16

K = 40
EPT = E // NW
NWIN = EPT // K
DEG_PAD = 30720
DEG_PT = DEG_PAD // NS
NP = 10240
ROWS_PT = NP // NS
CH = 32

_mesh = plsc.VectorSubcoreMesh(core_axis_name="c", subcore_axis_name="s")


@functools.partial(
    pl.kernel,
    out_type=jax.ShapeDtypeStruct((NC * DEG_PAD,), jnp.float32),
    mesh=_mesh,
    scratch_types=[
        pltpu.VMEM((K,), jnp.int32),
        pltpu.VMEM((K,), jnp.float32),
        pltpu.VMEM((DEG_PT,), jnp.float32),
        pltpu.VMEM_SHARED((DEG_PAD,), jnp.float32),
        pltpu.SemaphoreType.DMA,
    ],
)
def _deg_sc(dstb_hbm, out_hbm, idx_v, ones_v, stage_v, acc_sh, sem):
    c = lax.axis_index("c")
    s = lax.axis_index("s")
    w = c * NS + s
    for i in range(K // L):
        ones_v[pl.ds(i * L, L)] = jnp.full((L,), 1.0, jnp.float32)

    def zbody(i, carry):
        stage_v[pl.ds(i * L, L)] = jnp.zeros((L,), jnp.float32)
        return carry

    lax.fori_loop(0, DEG_PT // L, zbody, 0)
    pltpu.sync_copy(stage_v, acc_sh.at[pl.ds(s * DEG_PT, DEG_PT)])
    plsc.subcore_barrier()

    ept_all = NG * E // NW

    def body(win, carry):
        base = w * ept_all + win * K
        pltpu.sync_copy(dstb_hbm.at[pl.ds(base, K)], idx_v)
        pltpu.sync_copy(ones_v, acc_sh.at[idx_v], add=True)
        return carry

    lax.fori_loop(0, ept_all // K, body, 0)
    plsc.subcore_barrier()
    pltpu.sync_copy(acc_sh.at[pl.ds(s * DEG_PT, DEG_PT)], stage_v)
    pltpu.sync_copy(stage_v, out_hbm.at[pl.ds(c * DEG_PAD + s * DEG_PT, DEG_PT)])


NBUF = 2
NGRP = NWIN // NBUF


@functools.partial(
    pl.kernel,
    out_type=jax.ShapeDtypeStruct((NG, NC, NP, D), jnp.float32),
    mesh=_mesh,
    scratch_types=[
        pltpu.VMEM((EPT,), jnp.int32),
        pltpu.VMEM((EPT,), jnp.int32),
        [pltpu.VMEM((K, D), jnp.float32) for _ in range(NBUF)],
        pltpu.VMEM((CH, D), jnp.float32),
        pltpu.VMEM((CH, D), jnp.float32),
        pltpu.VMEM_SHARED((NP, D), jnp.float32),
        [pltpu.SemaphoreType.DMA for _ in range(NBUF)],
        [pltpu.SemaphoreType.DMA for _ in range(NBUF)],
    ],
)
def _agg3_sc(tables_hbm, src_hbm, dst_hbm, out_hbm,
             src_all, dst_all, bufs, zbuf, wbuf, acc_sh, gsems, ssems):
    c = lax.axis_index("c")
    s = lax.axis_index("s")
    w = c * NS + s

    def zb(i, carry):
        for j in range(D // L):
            zbuf[i, pl.ds(j * L, L)] = jnp.zeros((L,), jnp.float32)
        return carry

    lax.fori_loop(0, CH, zb, 0)
    for k in range(ROWS_PT // CH):
        pltpu.sync_copy(zbuf, acc_sh.at[pl.ds(s * ROWS_PT + k * CH, CH)])

    for g in range(NG):
        pltpu.sync_copy(src_hbm.at[pl.ds(g * E + w * EPT, EPT)], src_all)
        pltpu.sync_copy(dst_hbm.at[pl.ds(g * E + w * EPT, EPT)], dst_all)
        plsc.subcore_barrier()

        def g_issue(win, b, g=g):
            pltpu.async_copy(tables_hbm.at[g].at[src_all.at[pl.ds(win * K, K)]],
                             bufs[b], gsems[b])

        def g_wait(win, b, g=g):
            pltpu.make_async_copy(
                tables_hbm.at[g].at[src_all.at[pl.ds(win * K, K)]],
                bufs[b], gsems[b]).wait()

        def s_issue(win, b):
            pltpu.async_copy(bufs[b], acc_sh.at[dst_all.at[pl.ds(win * K, K)]],
                             ssems[b], add=True)

        def s_wait(win, b):
            pltpu.make_async_copy(bufs[b],
                                  acc_sh.at[dst_all.at[pl.ds(win * K, K)]],
                                  ssems[b]).wait()

        for b in range(NBUF):
            g_issue(b, b)

        def grp_body(grp, carry):
            for b in range(NBUF):
                win = grp * NBUF + b
                g_wait(win, b)
                s_issue(win, b)
                s_wait(win, b)
                g_issue(win + NBUF, b)
            return carry

        lax.fori_loop(0, NGRP - 1, grp_body, 0)
        for b in range(NBUF):
            win = (NGRP - 1) * NBUF + b
            g_wait(win, b)
            s_issue(win, b)
            s_wait(win, b)

        plsc.subcore_barrier()
        for k in range(ROWS_PT // CH):
            r0 = s * ROWS_PT + k * CH
            pltpu.sync_copy(acc_sh.at[pl.ds(r0, CH)], wbuf)
            pltpu.sync_copy(wbuf, out_hbm.at[g, c, pl.ds(r0, CH)])
            pltpu.sync_copy(zbuf, acc_sh.at[pl.ds(r0, CH)])


def _norm_kernel(deg_ref, o_ref):
    d = deg_ref[pl.ds(0, DEG_PAD)] + deg_ref[pl.ds(DEG_PAD, DEG_PAD)]
    o_ref[...] = lax.rsqrt(jnp.clip(d, 1.0, None))


def _norm(deg_partials):
    return pl.pallas_call(
        _norm_kernel,
        out_shape=jax.ShapeDtypeStruct((DEG_PAD,), jnp.float32),
    )(deg_partials)


def _mm1_kernel(x_ref, w_ref, b_ref, n_ref, o_ref):
    t0 = x_ref[...] @ w_ref[...] + b_ref[...]
    for g in range(NG):
        o_ref[g] = t0 * n_ref[g][:, None]


def _mm1(x, W1, b1, norm):
    return pl.pallas_call(
        _mm1_kernel,
        out_shape=jax.ShapeDtypeStruct((NG, N, D), jnp.float32),
    )(x, W1, b1, norm)


def _mm2_kernel(p_ref, w_ref, b_ref, n_ref, o_ref):
    ng = n_ref[0, 0][:, None]
    p = p_ref[0, 0][:N] + p_ref[0, 1][:N]
    h = jnp.maximum(p * ng, 0.0)
    o_ref[0] = (h @ w_ref[...] + b_ref[...]) * ng


def _mm2(partials, W2, b2, norm):
    return pl.pallas_call(
        _mm2_kernel,
        grid=(NG,),
        in_specs=[
            pl.BlockSpec((1, NC, NP, D), lambda g: (g, 0, 0, 0)),
            pl.BlockSpec((D, D), lambda g: (0, 0)),
            pl.BlockSpec((D,), lambda g: (0,)),
            pl.BlockSpec((1, 1, N), lambda g: (g, 0, 0)),
        ],
        out_specs=pl.BlockSpec((1, N, D), lambda g: (g, 0, 0)),
        out_shape=jax.ShapeDtypeStruct((NG, N, D), jnp.float32),
    )(partials, W2, b2, norm.reshape(NG, 1, N))


def _fuse_kernel(p_ref, n_ref, as_ref, ac_ref, wd_ref, bd_ref, o_ref):
    hs = []
    for g in range(NG):
        p = p_ref[g, 0][:N] + p_ref[g, 1][:N]
        hs.append(jnp.maximum(p * n_ref[g][:, None], 0.0))
    h, h1, h2 = hs
    l1 = jnp.sum(jnp.max(h1, axis=0) * as_ref[:, 0])
    l2 = jnp.sum(jnp.max(h2, axis=0) * as_ref[:, 0])
    m = jnp.maximum(l1, l2)
    e1 = jnp.exp(l1 - m)
    e2 = jnp.exp(l2 - m)
    h_homo = h1 * (e1 / (e1 + e2)) + h2 * (e2 / (e1 + e2))
    c1 = jnp.sum(jnp.max(h, axis=0) * ac_ref[:, 0])
    c2 = jnp.sum(jnp.max(h_homo, axis=0) * ac_ref[:, 0])
    mc = jnp.maximum(c1, c2)
    f1 = jnp.exp(c1 - mc)
    f2 = jnp.exp(c2 - mc)
    hf = h * (f1 / (f1 + f2)) + h_homo * (f2 / (f1 + f2))
    o_ref[...] = hf @ wd_ref[...] + bd_ref[...]


def _fuse(partials, norm, attn_s, attn_c, W_dec, b_dec):
    return pl.pallas_call(
        _fuse_kernel,
        out_shape=jax.ShapeDtypeStruct((N, NCLS), jnp.float32),
    )(partials, norm, attn_s, attn_c, W_dec, b_dec)


def kernel(features, edge_index, edge_index_homo1, edge_index_homo2,
           W1, b1, W2, b2, W_dec, b_dec, attn_simplex, attn_complex):
    src = jnp.stack([edge_index[0], edge_index_homo1[0], edge_index_homo2[0]])
    dst = jnp.stack([edge_index[1], edge_index_homo1[1], edge_index_homo2[1]])
    bias = (jnp.arange(NG, dtype=jnp.int32) * N)[:, None]
    dstb = (dst + bias).reshape(-1)
    deg_p = _deg_sc(dstb)
    norm_flat = _norm(deg_p)
    norm = norm_flat[:NG * N].reshape(NG, N)

    src_f = src.reshape(-1)
    dst_f = dst.reshape(-1)
    tables1 = _mm1(features, W1, b1, norm)
    p1 = _agg3_sc(tables1, src_f, dst_f)
    tables2 = _mm2(p1, W2, b2, norm)
    p2 = _agg3_sc(tables2, src_f, dst_f)
    return _fuse(p2, norm, attn_simplex, attn_complex, W_dec, b_dec)


# speedup vs baseline: 19.3408x; 1.3510x over previous
import functools
import jax
import jax.numpy as jnp
from jax import lax
from jax.experimental import pallas as pl
from jax.experimental.pallas import tpu as pltpu
from jax.experimental.pallas import tpu_sc as plsc

N = 10000
E = 320000
D = 128
NCLS = 16
NG = 3

NC = 2
NS = 16
NW = NC * NS
L = 16

NP = 10240
EPT = E // NW
EPT_PAD = NP
K = 128
NWIN = EPT_PAD // K
NHALF = 2
HWIN = NWIN // NHALF
HEDGE = HWIN * K
NBUF = 2
NGRP = HWIN // NBUF

DEG_PAD = 30720
DEG_PT = DEG_PAD // NS
DK = 48
ROWS_PT = NP // NS

_mesh = plsc.VectorSubcoreMesh(core_axis_name="c", subcore_axis_name="s")


@functools.partial(
    pl.kernel,
    out_type=jax.ShapeDtypeStruct((NC * DEG_PAD,), jnp.float32),
    mesh=_mesh,
    scratch_types=[
        pltpu.VMEM((DK,), jnp.int32),
        pltpu.VMEM((DK,), jnp.float32),
        pltpu.VMEM((DEG_PT,), jnp.float32),
        pltpu.VMEM_SHARED((DEG_PAD,), jnp.float32),
        pltpu.SemaphoreType.DMA,
    ],
)
def _deg_sc(dstb_hbm, out_hbm, idx_v, ones_v, stage_v, acc_sh, sem):
    c = lax.axis_index("c")
    s = lax.axis_index("s")
    w = c * NS + s
    for i in range(DK // L):
        ones_v[pl.ds(i * L, L)] = jnp.full((L,), 1.0, jnp.float32)

    def zbody(i, carry):
        stage_v[pl.ds(i * L, L)] = jnp.zeros((L,), jnp.float32)
        return carry

    lax.fori_loop(0, DEG_PT // L, zbody, 0)
    pltpu.sync_copy(stage_v, acc_sh.at[pl.ds(s * DEG_PT, DEG_PT)])
    plsc.subcore_barrier()

    ept_all = NG * E // NW

    def body(win, carry):
        base = w * ept_all + win * DK
        pltpu.sync_copy(dstb_hbm.at[pl.ds(base, DK)], idx_v)
        pltpu.sync_copy(ones_v, acc_sh.at[idx_v], add=True)
        return carry

    lax.fori_loop(0, ept_all // DK, body, 0)
    plsc.subcore_barrier()
    pltpu.sync_copy(acc_sh.at[pl.ds(s * DEG_PT, DEG_PT)], stage_v)
    pltpu.sync_copy(stage_v, out_hbm.at[pl.ds(c * DEG_PAD + s * DEG_PT, DEG_PT)])


@functools.partial(
    pl.kernel,
    out_type=jax.ShapeDtypeStruct((NG, NC, NP, D), jnp.float32),
    mesh=_mesh,
    scratch_types=[
        pltpu.VMEM((HEDGE,), jnp.int32),
        pltpu.VMEM((HEDGE,), jnp.int32),
        [pltpu.VMEM((K,), jnp.int32) for _ in range(NBUF)],
        [pltpu.VMEM((K,), jnp.int32) for _ in range(NBUF)],
        [pltpu.VMEM((K, D), jnp.float32) for _ in range(NBUF)],
        pltpu.VMEM_SHARED((NP, D), jnp.float32),
        [pltpu.SemaphoreType.DMA for _ in range(NBUF)],
        [pltpu.SemaphoreType.DMA for _ in range(NBUF)],
    ],
)
def _agg3_sc(tables_hbm, src_hbm, dst_hbm, out_hbm,
             src_all, dst_all, idx_s, idx_d, bufs, acc_sh, gsems, ssems):
    c = lax.axis_index("c")
    s = lax.axis_index("s")
    w = c * NS + s

    def zero_buf(buf):
        def zb(i, carry):
            for j in range(D // L):
                buf[i, pl.ds(j * L, L)] = jnp.zeros((L,), jnp.float32)
            return carry
        lax.fori_loop(0, K, zb, 0)

    zero_buf(bufs[1])
    for k in range(ROWS_PT // K):
        pltpu.sync_copy(bufs[1], acc_sh.at[pl.ds(s * ROWS_PT + k * K, K)])

    for g in range(NG):
        plsc.subcore_barrier()
        for h in range(NHALF):
            base = (g * NW + w) * EPT_PAD + h * HEDGE
            pltpu.sync_copy(src_hbm.at[pl.ds(base, HEDGE)], src_all)
            pltpu.sync_copy(dst_hbm.at[pl.ds(base, HEDGE)], dst_all)

            def mat_s(win, b):
                for j in range(K // L):
                    idx_s[b][pl.ds(j * L, L)] = src_all[pl.ds(win * K + j * L, L)]

            def mat_d(win, b):
                for j in range(K // L):
                    idx_d[b][pl.ds(j * L, L)] = dst_all[pl.ds(win * K + j * L, L)]

            def g_issue(b, g=g):
                pltpu.async_copy(tables_hbm.at[g].at[idx_s[b]],
                                 bufs[b], gsems[b])

            def g_wait(b, g=g):
                pltpu.make_async_copy(tables_hbm.at[g].at[idx_s[b]],
                                      bufs[b], gsems[b]).wait()

            def s_issue(b):
                pltpu.async_copy(bufs[b], acc_sh.at[idx_d[b]],
                                 ssems[b], add=True)

            def s_wait(b):
                pltpu.make_async_copy(bufs[b], acc_sh.at[idx_d[b]],
                                      ssems[b]).wait()

            for b in range(NBUF):
                mat_s(b, b)
                g_issue(b)

            def grp_body(grp, carry):
                for b in range(NBUF):
                    win = grp * NBUF + b
                    g_wait(b)
                    mat_d(win, b)
                    s_issue(b)
                    s_wait(b)
                    mat_s(win + NBUF, b)
                    g_issue(b)
                return carry

            lax.fori_loop(0, NGRP - 1, grp_body, 0)
            for b in range(NBUF):
                win = (NGRP - 1) * NBUF + b
                g_wait(b)
                mat_d(win, b)
                s_issue(b)
                s_wait(b)

        plsc.subcore_barrier()
        zero_buf(bufs[1])
        for k in range(ROWS_PT // K):
            r0 = s * ROWS_PT + k * K
            pltpu.sync_copy(acc_sh.at[pl.ds(r0, K)], bufs[0])
            pltpu.sync_copy(bufs[0], out_hbm.at[g, c, pl.ds(r0, K)])
            pltpu.sync_copy(bufs[1], acc_sh.at[pl.ds(r0, K)])


def _norm_kernel(deg_ref, o_ref):
    d = deg_ref[pl.ds(0, DEG_PAD)] + deg_ref[pl.ds(DEG_PAD, DEG_PAD)]
    o_ref[...] = lax.rsqrt(jnp.clip(d, 1.0, None))


def _norm(deg_partials):
    return pl.pallas_call(
        _norm_kernel,
        out_shape=jax.ShapeDtypeStruct((DEG_PAD,), jnp.float32),
    )(deg_partials)


def _mm1_kernel(x_ref, w_ref, b_ref, n_ref, o_ref):
    t0 = x_ref[...] @ w_ref[...] + b_ref[...]
    for g in range(NG):
        o_ref[g] = t0 * n_ref[g][:, None]


def _mm1(x_pad, W1, b1, norm):
    return pl.pallas_call(
        _mm1_kernel,
        out_shape=jax.ShapeDtypeStruct((NG, NP, D), jnp.float32),
    )(x_pad, W1, b1, norm)


def _mm2_kernel(p_ref, w_ref, b_ref, n_ref, o_ref):
    ng = n_ref[0, 0][:, None]
    h = jnp.maximum((p_ref[0, 0] + p_ref[0, 1]) * ng, 0.0)
    o_ref[0] = (h @ w_ref[...] + b_ref[...]) * ng


def _mm2(partials, W2, b2, norm):
    return pl.pallas_call(
        _mm2_kernel,
        grid=(NG,),
        in_specs=[
            pl.BlockSpec((1, NC, NP, D), lambda g: (g, 0, 0, 0)),
            pl.BlockSpec((D, D), lambda g: (0, 0)),
            pl.BlockSpec((D,), lambda g: (0,)),
            pl.BlockSpec((1, 1, NP), lambda g: (g, 0, 0)),
        ],
        out_specs=pl.BlockSpec((1, NP, D), lambda g: (g, 0, 0)),
        out_shape=jax.ShapeDtypeStruct((NG, NP, D), jnp.float32),
    )(partials, W2, b2, norm.reshape(NG, 1, NP))


def _fuse_kernel(p_ref, n_ref, as_ref, ac_ref, wd_ref, bd_ref, o_ref):
    hs = []
    for g in range(NG):
        p = p_ref[g, 0][:N] + p_ref[g, 1][:N]
        hs.append(jnp.maximum(p * n_ref[g][:N, None], 0.0))
    h, h1, h2 = hs
    l1 = jnp.sum(jnp.max(h1, axis=0) * as_ref[:, 0])
    l2 = jnp.sum(jnp.max(h2, axis=0) * as_ref[:, 0])
    m = jnp.maximum(l1, l2)
    e1 = jnp.exp(l1 - m)
    e2 = jnp.exp(l2 - m)
    h_homo = h1 * (e1 / (e1 + e2)) + h2 * (e2 / (e1 + e2))
    c1 = jnp.sum(jnp.max(h, axis=0) * ac_ref[:, 0])
    c2 = jnp.sum(jnp.max(h_homo, axis=0) * ac_ref[:, 0])
    mc = jnp.maximum(c1, c2)
    f1 = jnp.exp(c1 - mc)
    f2 = jnp.exp(c2 - mc)
    hf = h * (f1 / (f1 + f2)) + h_homo * (f2 / (f1 + f2))
    o_ref[...] = hf @ wd_ref[...] + bd_ref[...]


def _fuse(partials, norm, attn_s, attn_c, W_dec, b_dec):
    return pl.pallas_call(
        _fuse_kernel,
        out_shape=jax.ShapeDtypeStruct((N, NCLS), jnp.float32),
    )(partials, norm, attn_s, attn_c, W_dec, b_dec)


def kernel(features, edge_index, edge_index_homo1, edge_index_homo2,
           W1, b1, W2, b2, W_dec, b_dec, attn_simplex, attn_complex):
    src = jnp.stack([edge_index[0], edge_index_homo1[0], edge_index_homo2[0]])
    dst = jnp.stack([edge_index[1], edge_index_homo1[1], edge_index_homo2[1]])
    pad_idx = (N + (jnp.arange(EPT_PAD - EPT, dtype=jnp.int32) % (NP - N)))
    pad3 = jnp.broadcast_to(pad_idx, (NG, NW, EPT_PAD - EPT))
    src_p = jnp.concatenate([src.reshape(NG, NW, EPT), pad3], axis=2)
    dst_p = jnp.concatenate([dst.reshape(NG, NW, EPT), pad3], axis=2)

    bias = (jnp.arange(NG, dtype=jnp.int32) * N)[:, None]
    dstb = (dst + bias).reshape(-1)
    deg_p = _deg_sc(dstb)
    norm_flat = _norm(deg_p)
    norm = jnp.pad(norm_flat[:NG * N].reshape(NG, N), ((0, 0), (0, NP - N)))

    x_pad = jnp.pad(features, ((0, NP - N), (0, 0)))
    src_f = src_p.reshape(-1)
    dst_f = dst_p.reshape(-1)
    tables1 = _mm1(x_pad, W1, b1, norm)
    p1 = _agg3_sc(tables1, src_f, dst_f)
    tables2 = _mm2(p1, W2, b2, norm)
    p2 = _agg3_sc(tables2, src_f, dst_f)
    return _fuse(p2, norm, attn_simplex, attn_complex, W_dec, b_dec)


# speedup vs baseline: 26.4240x; 1.3662x over previous
import functools
import jax
import jax.numpy as jnp
from jax import lax
from jax.experimental import pallas as pl
from jax.experimental.pallas import tpu as pltpu
from jax.experimental.pallas import tpu_sc as plsc

N = 10000
E = 320000
D = 128
NCLS = 16
NG = 3

NC = 2
NS = 16
NW = NC * NS
L = 16

NP = 10240
EPT = E // NW
EPT_PAD = NP
K = 128
NWIN = EPT_PAD // K
NHALF = 2
HWIN = NWIN // NHALF
HEDGE = HWIN * K
NBUF = 2
NGRP = HWIN // NBUF

DEG_PAD = 30720
DEG_PT = DEG_PAD // NS
DK = 240
DEPT = NG * E // NW
DWIN = DEPT // DK
ROWS_PT = NP // NS

_mesh = plsc.VectorSubcoreMesh(core_axis_name="c", subcore_axis_name="s")


@functools.partial(
    pl.kernel,
    out_type=jax.ShapeDtypeStruct((NC * DEG_PAD,), jnp.float32),
    mesh=_mesh,
    scratch_types=[
        pltpu.VMEM((DEPT,), jnp.int32),
        [pltpu.VMEM((DK,), jnp.int32) for _ in range(2)],
        pltpu.VMEM((DK,), jnp.float32),
        pltpu.VMEM((DEG_PT,), jnp.float32),
        pltpu.VMEM_SHARED((DEG_PAD,), jnp.float32),
        [pltpu.SemaphoreType.DMA for _ in range(2)],
    ],
)
def _deg_sc(dstb_hbm, out_hbm, idx_all, idx_v, ones_v, stage_v, acc_sh, ssems):
    c = lax.axis_index("c")
    s = lax.axis_index("s")
    w = c * NS + s
    for i in range(DK // L):
        ones_v[pl.ds(i * L, L)] = jnp.full((L,), 1.0, jnp.float32)

    def zbody(i, carry):
        stage_v[pl.ds(i * L, L)] = jnp.zeros((L,), jnp.float32)
        return carry

    lax.fori_loop(0, DEG_PT // L, zbody, 0)
    pltpu.sync_copy(dstb_hbm.at[pl.ds(w * DEPT, DEPT)], idx_all)
    pltpu.sync_copy(stage_v, acc_sh.at[pl.ds(s * DEG_PT, DEG_PT)])
    plsc.subcore_barrier()

    def mat(win, b):
        for j in range(DK // L):
            idx_v[b][pl.ds(j * L, L)] = idx_all[pl.ds(win * DK + j * L, L)]

    def s_issue(b):
        pltpu.async_copy(ones_v, acc_sh.at[idx_v[b]], ssems[b], add=True)

    def s_wait(b):
        pltpu.make_async_copy(ones_v, acc_sh.at[idx_v[b]], ssems[b]).wait()

    for b in range(2):
        mat(b, b)
        s_issue(b)

    def grp_body(grp, carry):
        for b in range(2):
            win = 2 + grp * 2 + b
            s_wait(b)
            mat(win, b)
            s_issue(b)
        return carry

    lax.fori_loop(0, (DWIN - 2) // 2, grp_body, 0)
    s_wait(0)
    mat(DWIN - 1, 0)
    s_issue(0)
    s_wait(0)
    s_wait(1)
    plsc.subcore_barrier()
    pltpu.sync_copy(acc_sh.at[pl.ds(s * DEG_PT, DEG_PT)], stage_v)
    pltpu.sync_copy(stage_v, out_hbm.at[pl.ds(c * DEG_PAD + s * DEG_PT, DEG_PT)])


@functools.partial(
    pl.kernel,
    out_type=jax.ShapeDtypeStruct((NG, NC, NP, D), jnp.float32),
    mesh=_mesh,
    scratch_types=[
        pltpu.VMEM((HEDGE,), jnp.int32),
        pltpu.VMEM((HEDGE,), jnp.int32),
        [pltpu.VMEM((K,), jnp.int32) for _ in range(NBUF)],
        [pltpu.VMEM((K,), jnp.int32) for _ in range(NBUF)],
        [pltpu.VMEM((K, D), jnp.float32) for _ in range(NBUF)],
        pltpu.VMEM_SHARED((NP, D), jnp.float32),
        [pltpu.SemaphoreType.DMA for _ in range(NBUF)],
        [pltpu.SemaphoreType.DMA for _ in range(NBUF)],
    ],
)
def _agg3_sc(tables_hbm, src_hbm, dst_hbm, out_hbm,
             src_all, dst_all, idx_s, idx_d, bufs, acc_sh, gsems, ssems):
    c = lax.axis_index("c")
    s = lax.axis_index("s")
    w = c * NS + s

    def zero_buf(buf):
        def zb(i, carry):
            for j in range(D // L):
                buf[i, pl.ds(j * L, L)] = jnp.zeros((L,), jnp.float32)
            return carry
        lax.fori_loop(0, K, zb, 0)

    zero_buf(bufs[1])
    for k in range(ROWS_PT // K):
        pltpu.sync_copy(bufs[1], acc_sh.at[pl.ds(s * ROWS_PT + k * K, K)])

    for g in range(NG):
        plsc.subcore_barrier()
        for h in range(NHALF):
            base = (g * NW + w) * EPT_PAD + h * HEDGE
            pltpu.sync_copy(src_hbm.at[pl.ds(base, HEDGE)], src_all)
            pltpu.sync_copy(dst_hbm.at[pl.ds(base, HEDGE)], dst_all)

            def mat_s(win, b):
                for j in range(K // L):
                    idx_s[b][pl.ds(j * L, L)] = src_all[pl.ds(win * K + j * L, L)]

            def mat_d(win, b):
                for j in range(K // L):
                    idx_d[b][pl.ds(j * L, L)] = dst_all[pl.ds(win * K + j * L, L)]

            def g_issue(b, g=g):
                pltpu.async_copy(tables_hbm.at[g].at[idx_s[b]],
                                 bufs[b], gsems[b])

            def g_wait(b, g=g):
                pltpu.make_async_copy(tables_hbm.at[g].at[idx_s[b]],
                                      bufs[b], gsems[b]).wait()

            def s_issue(b):
                pltpu.async_copy(bufs[b], acc_sh.at[idx_d[b]],
                                 ssems[b], add=True)

            def s_wait(b):
                pltpu.make_async_copy(bufs[b], acc_sh.at[idx_d[b]],
                                      ssems[b]).wait()

            for b in range(NBUF):
                mat_s(b, b)
                g_issue(b)

            def grp_body(grp, carry):
                for b in range(NBUF):
                    win = grp * NBUF + b
                    g_wait(b)
                    mat_d(win, b)
                    s_issue(b)
                    s_wait(b)
                    mat_s(win + NBUF, b)
                    g_issue(b)
                return carry

            lax.fori_loop(0, NGRP - 1, grp_body, 0)
            for b in range(NBUF):
                win = (NGRP - 1) * NBUF + b
                g_wait(b)
                mat_d(win, b)
                s_issue(b)
                s_wait(b)

        plsc.subcore_barrier()
        zero_buf(bufs[1])
        for k in range(ROWS_PT // K):
            r0 = s * ROWS_PT + k * K
            pltpu.sync_copy(acc_sh.at[pl.ds(r0, K)], bufs[0])
            pltpu.sync_copy(bufs[0], out_hbm.at[g, c, pl.ds(r0, K)])
            pltpu.sync_copy(bufs[1], acc_sh.at[pl.ds(r0, K)])


def _norm_kernel(deg_ref, o_ref):
    d = deg_ref[pl.ds(0, DEG_PAD)] + deg_ref[pl.ds(DEG_PAD, DEG_PAD)]
    o_ref[...] = lax.rsqrt(jnp.clip(d, 1.0, None))


def _norm(deg_partials):
    return pl.pallas_call(
        _norm_kernel,
        out_shape=jax.ShapeDtypeStruct((DEG_PAD,), jnp.float32),
    )(deg_partials)


def _mm1_kernel(x_ref, w_ref, b_ref, n_ref, o_ref):
    t0 = x_ref[...] @ w_ref[...] + b_ref[...]
    for g in range(NG):
        o_ref[g] = t0 * n_ref[g][:, None]


def _mm1(x_pad, W1, b1, norm):
    return pl.pallas_call(
        _mm1_kernel,
        out_shape=jax.ShapeDtypeStruct((NG, NP, D), jnp.float32),
    )(x_pad, W1, b1, norm)


def _mm2_kernel(p_ref, w_ref, b_ref, n_ref, o_ref):
    ng = n_ref[0, 0][:, None]
    h = jnp.maximum((p_ref[0, 0] + p_ref[0, 1]) * ng, 0.0)
    o_ref[0] = (h @ w_ref[...] + b_ref[...]) * ng


def _mm2(partials, W2, b2, norm):
    return pl.pallas_call(
        _mm2_kernel,
        grid=(NG,),
        in_specs=[
            pl.BlockSpec((1, NC, NP, D), lambda g: (g, 0, 0, 0)),
            pl.BlockSpec((D, D), lambda g: (0, 0)),
            pl.BlockSpec((D,), lambda g: (0,)),
            pl.BlockSpec((1, 1, NP), lambda g: (g, 0, 0)),
        ],
        out_specs=pl.BlockSpec((1, NP, D), lambda g: (g, 0, 0)),
        out_shape=jax.ShapeDtypeStruct((NG, NP, D), jnp.float32),
    )(partials, W2, b2, norm.reshape(NG, 1, NP))


def _fuse_kernel(p_ref, n_ref, as_ref, ac_ref, wd_ref, bd_ref, o_ref):
    hs = []
    for g in range(NG):
        p = p_ref[g, 0][:N] + p_ref[g, 1][:N]
        hs.append(jnp.maximum(p * n_ref[g][:N, None], 0.0))
    h, h1, h2 = hs
    l1 = jnp.sum(jnp.max(h1, axis=0) * as_ref[:, 0])
    l2 = jnp.sum(jnp.max(h2, axis=0) * as_ref[:, 0])
    m = jnp.maximum(l1, l2)
    e1 = jnp.exp(l1 - m)
    e2 = jnp.exp(l2 - m)
    h_homo = h1 * (e1 / (e1 + e2)) + h2 * (e2 / (e1 + e2))
    c1 = jnp.sum(jnp.max(h, axis=0) * ac_ref[:, 0])
    c2 = jnp.sum(jnp.max(h_homo, axis=0) * ac_ref[:, 0])
    mc = jnp.maximum(c1, c2)
    f1 = jnp.exp(c1 - mc)
    f2 = jnp.exp(c2 - mc)
    hf = h * (f1 / (f1 + f2)) + h_homo * (f2 / (f1 + f2))
    o_ref[...] = hf @ wd_ref[...] + bd_ref[...]


def _fuse(partials, norm, attn_s, attn_c, W_dec, b_dec):
    return pl.pallas_call(
        _fuse_kernel,
        out_shape=jax.ShapeDtypeStruct((N, NCLS), jnp.float32),
    )(partials, norm, attn_s, attn_c, W_dec, b_dec)


def kernel(features, edge_index, edge_index_homo1, edge_index_homo2,
           W1, b1, W2, b2, W_dec, b_dec, attn_simplex, attn_complex):
    src = jnp.stack([edge_index[0], edge_index_homo1[0], edge_index_homo2[0]])
    dst = jnp.stack([edge_index[1], edge_index_homo1[1], edge_index_homo2[1]])
    pad_idx = (N + (jnp.arange(EPT_PAD - EPT, dtype=jnp.int32) % (NP - N)))
    pad3 = jnp.broadcast_to(pad_idx, (NG, NW, EPT_PAD - EPT))
    src_p = jnp.concatenate([src.reshape(NG, NW, EPT), pad3], axis=2)
    dst_p = jnp.concatenate([dst.reshape(NG, NW, EPT), pad3], axis=2)

    bias = (jnp.arange(NG, dtype=jnp.int32) * N)[:, None]
    dstb = (dst + bias).reshape(-1)
    deg_p = _deg_sc(dstb)
    norm_flat = _norm(deg_p)
    norm = jnp.pad(norm_flat[:NG * N].reshape(NG, N), ((0, 0), (0, NP - N)))

    x_pad = jnp.pad(features, ((0, NP - N), (0, 0)))
    src_f = src_p.reshape(-1)
    dst_f = dst_p.reshape(-1)
    tables1 = _mm1(x_pad, W1, b1, norm)
    p1 = _agg3_sc(tables1, src_f, dst_f)
    tables2 = _mm2(p1, W2, b2, norm)
    p2 = _agg3_sc(tables2, src_f, dst_f)
    return _fuse(p2, norm, attn_simplex, attn_complex, W_dec, b_dec)
